# baseline (speedup 1.0000x reference)
import functools

import numpy as np
import jax
import jax.numpy as jnp
from jax import lax
from jax.experimental import pallas as pl
from jax.experimental.pallas import tpu as pltpu
from jax.experimental.pallas import tpu_sc as plsc

VOCAB = 64
SEQ = 512
D = 128
H = 768

NUM_CORES = 2
NUM_SUBCORES = 16
NW = NUM_CORES * NUM_SUBCORES

LANES = 16
LPW = SEQ // NW


def _pe_flat():
    pos = np.arange(SEQ)[:, None].astype(np.float32)
    i = np.arange(D // 2)[None, :].astype(np.float32)
    ang = pos / np.power(10000.0, (2.0 * i) / float(D))
    pe = np.zeros((SEQ, D), dtype=np.float32)
    pe[:, 0::2] = np.sin(ang)
    pe[:, 1::2] = np.cos(ang)
    return pe.reshape(-1)


_PE_CONST = _pe_flat()


def _matmul_body(a_ref, w_ref, b_ref, o_ref):
    o_ref[...] = (
        jnp.dot(a_ref[...], w_ref[...], preferred_element_type=jnp.float32)
        + b_ref[...]
    )


def _reduced_table(matbert_table, W, b):
    return pl.pallas_call(
        _matmul_body,
        out_shape=jax.ShapeDtypeStruct((VOCAB, D), jnp.float32),
    )(matbert_table, W, b.reshape(1, D))


def _make_sc_kernel(batch):
    bblk = 16
    pblk = LPW
    nbblk = batch // bblk
    npblk = LPW // pblk
    nchunk = npblk * nbblk
    ncol = D // LANES

    mesh = plsc.VectorSubcoreMesh(
        core_axis_name="c",
        subcore_axis_name="s",
        num_cores=NUM_CORES,
        num_subcores=NUM_SUBCORES,
    )

    @functools.partial(
        pl.kernel,
        out_type=jax.ShapeDtypeStruct((batch * SEQ * D,), jnp.float32),
        mesh=mesh,
        compiler_params=pltpu.CompilerParams(needs_layout_passes=False),
        scratch_types=[
            pltpu.VMEM((VOCAB * D,), jnp.float32),
            pltpu.VMEM((LPW * D,), jnp.float32),
            pltpu.VMEM((LPW, batch), jnp.int32),
            pltpu.VMEM((bblk * pblk * D,), jnp.float32),
            pltpu.VMEM((bblk * pblk * D,), jnp.float32),
            pltpu.SemaphoreType.DMA,
            pltpu.SemaphoreType.DMA,
        ],
    )
    def sc_gather(
        xt_hbm, red_hbm, pe_hbm, out_hbm, tab_v, pe_v, tok_v, buf0, buf1,
        sem0, sem1,
    ):
        wid = lax.axis_index("s") * NUM_CORES + lax.axis_index("c")
        l0 = wid * LPW
        pltpu.sync_copy(red_hbm, tab_v)
        pltpu.sync_copy(pe_hbm.at[pl.ds(l0 * D, LPW * D)], pe_v)
        pltpu.sync_copy(xt_hbm.at[pl.ds(l0, LPW)], tok_v)

        iota = lax.iota(jnp.int32, 16)

        def compute_chunk(q, buf):
            sb = lax.div(q, nbblk)
            b0 = lax.rem(q, nbblk) * bblk

            def pos_body(j, _):
                dl = sb * pblk + j
                pes = [
                    pe_v[pl.ds(dl * D + c * LANES, 16)] for c in range(ncol)
                ]

                @plsc.parallel_loop(0, bblk, step=16)
                def _(g0):
                    toks16 = tok_v[dl, pl.ds(b0 + g0, 16)] * D

                    def row_loads(rr):
                        rb = jnp.broadcast_to(toks16[rr], (16,)) + iota
                        return [
                            plsc.load_gather(
                                tab_v.at[pl.ds(c * LANES, VOCAB * D - c * LANES)],
                                [rb],
                            )
                            for c in range(ncol)
                        ]

                    def st(rr, c, val):
                        base = ((g0 + rr) * pblk + j) * D
                        buf[pl.ds(base + c * LANES, 16)] = val + pes[c]

                    staged = row_loads(0)
                    for rr in range(1, 16):
                        rb = jnp.broadcast_to(toks16[rr], (16,)) + iota
                        nxt = []
                        for c in range(ncol):
                            nxt.append(plsc.load_gather(
                                tab_v.at[pl.ds(c * LANES, VOCAB * D - c * LANES)],
                                [rb],
                            ))
                            st(rr - 1, c, staged[c])
                        staged = nxt
                    for c in range(ncol):
                        st(15, c, staged[c])

                return 0

            lax.fori_loop(0, pblk, pos_body, 0)

        def start_out(q, buf, sem):
            sb = lax.div(q, nbblk)
            b0 = lax.rem(q, nbblk) * bblk
            l8 = l0 + sb * pblk
            for i in range(bblk):
                pltpu.async_copy(
                    buf.at[pl.ds(i * pblk * D, pblk * D)],
                    out_hbm.at[pl.ds(((b0 + i) * SEQ + l8) * D, pblk * D)],
                    sem,
                )

        def drain_out(buf, sem):
            pltpu.make_async_copy(
                buf, out_hbm.at[pl.ds(0, bblk * pblk * D)], sem
            ).wait()

        compute_chunk(0, buf0)
        start_out(0, buf0, sem0)
        compute_chunk(1, buf1)
        start_out(1, buf1, sem1)

        def pair(kk, _):
            q0 = kk * 2
            q1 = q0 + 1
            drain_out(buf0, sem0)
            compute_chunk(q0, buf0)
            start_out(q0, buf0, sem0)
            drain_out(buf1, sem1)
            compute_chunk(q1, buf1)
            start_out(q1, buf1, sem1)
            return 0

        lax.fori_loop(1, nchunk // 2, pair, 0)
        drain_out(buf0, sem0)
        drain_out(buf1, sem1)

    return sc_gather


def kernel(x, matbert_table, W, b):
    batch, seq = x.shape
    reduced = _reduced_table(matbert_table, W, b)
    pe = jnp.asarray(_PE_CONST)
    sc = _make_sc_kernel(batch)
    return sc(x.T, reduced.reshape(-1), pe).reshape(batch, seq, D)


# speedup vs baseline: 1.0153x; 1.0153x over previous
import functools

import numpy as np
import jax
import jax.numpy as jnp
from jax import lax
from jax.experimental import pallas as pl
from jax.experimental.pallas import tpu as pltpu
from jax.experimental.pallas import tpu_sc as plsc

VOCAB = 64
SEQ = 512
D = 128
H = 768

NUM_CORES = 2
NUM_SUBCORES = 16
NW = NUM_CORES * NUM_SUBCORES

LANES = 16
LPW = SEQ // NW


def _pe_flat():
    pos = np.arange(SEQ)[:, None].astype(np.float32)
    i = np.arange(D // 2)[None, :].astype(np.float32)
    ang = pos / np.power(10000.0, (2.0 * i) / float(D))
    pe = np.zeros((SEQ, D), dtype=np.float32)
    pe[:, 0::2] = np.sin(ang)
    pe[:, 1::2] = np.cos(ang)
    return pe.reshape(-1)


_PE_CONST = _pe_flat()


def _matmul_body(a_ref, w_ref, b_ref, o_ref):
    o_ref[...] = (
        jnp.dot(a_ref[...], w_ref[...], preferred_element_type=jnp.float32)
        + b_ref[...]
    )


def _reduced_table(matbert_table, W, b):
    return pl.pallas_call(
        _matmul_body,
        out_shape=jax.ShapeDtypeStruct((VOCAB, D), jnp.float32),
    )(matbert_table, W, b.reshape(1, D))


def _make_sc_kernel(batch):
    bblk = 16
    pblk = LPW
    nbblk = batch // bblk
    npblk = LPW // pblk
    nchunk = npblk * nbblk
    ncol = D // LANES

    mesh = plsc.VectorSubcoreMesh(
        core_axis_name="c",
        subcore_axis_name="s",
        num_cores=NUM_CORES,
        num_subcores=NUM_SUBCORES,
    )

    @functools.partial(
        pl.kernel,
        out_type=jax.ShapeDtypeStruct((batch * SEQ * D,), jnp.float32),
        mesh=mesh,
        compiler_params=pltpu.CompilerParams(needs_layout_passes=False),
        scratch_types=[
            pltpu.VMEM((VOCAB * D,), jnp.float32),
            pltpu.VMEM((LPW * D,), jnp.float32),
            pltpu.VMEM((LPW, batch), jnp.int32),
            pltpu.VMEM((bblk * pblk * D,), jnp.float32),
            pltpu.VMEM((bblk * pblk * D,), jnp.float32),
            pltpu.VMEM((bblk * pblk * D,), jnp.float32),
            pltpu.SemaphoreType.DMA,
            pltpu.SemaphoreType.DMA,
            pltpu.SemaphoreType.DMA,
        ],
    )
    def sc_gather(
        xt_hbm, red_hbm, pe_hbm, out_hbm, tab_v, pe_v, tok_v, buf0, buf1,
        buf2, sem0, sem1, sem2,
    ):
        wid = lax.axis_index("s") * NUM_CORES + lax.axis_index("c")
        l0 = wid * LPW
        pltpu.sync_copy(red_hbm, tab_v)
        pltpu.sync_copy(pe_hbm.at[pl.ds(l0 * D, LPW * D)], pe_v)
        pltpu.sync_copy(xt_hbm.at[pl.ds(l0, LPW)], tok_v)

        iota = lax.iota(jnp.int32, 16)

        def compute_chunk(q, buf):
            sb = lax.div(q, nbblk)
            b0 = lax.rem(q, nbblk) * bblk

            def pos_body(j, _):
                dl = sb * pblk + j
                pes = [
                    pe_v[pl.ds(dl * D + c * LANES, 16)] for c in range(ncol)
                ]

                @plsc.parallel_loop(0, bblk, step=16)
                def _(g0):
                    toks16 = tok_v[dl, pl.ds(b0 + g0, 16)] * D

                    def row_loads(rr):
                        rb = jnp.broadcast_to(toks16[rr], (16,)) + iota
                        return [
                            plsc.load_gather(
                                tab_v.at[pl.ds(c * LANES, VOCAB * D - c * LANES)],
                                [rb],
                            )
                            for c in range(ncol)
                        ]

                    def st(rr, c, val):
                        base = ((g0 + rr) * pblk + j) * D
                        buf[pl.ds(base + c * LANES, 16)] = val + pes[c]

                    staged = row_loads(0)
                    for rr in range(1, 16):
                        rb = jnp.broadcast_to(toks16[rr], (16,)) + iota
                        nxt = []
                        for c in range(ncol):
                            nxt.append(plsc.load_gather(
                                tab_v.at[pl.ds(c * LANES, VOCAB * D - c * LANES)],
                                [rb],
                            ))
                            st(rr - 1, c, staged[c])
                        staged = nxt
                    for c in range(ncol):
                        st(15, c, staged[c])

                return 0

            lax.fori_loop(0, pblk, pos_body, 0)

        def start_out(q, buf, sem):
            sb = lax.div(q, nbblk)
            b0 = lax.rem(q, nbblk) * bblk
            l8 = l0 + sb * pblk
            for i in range(bblk):
                pltpu.async_copy(
                    buf.at[pl.ds(i * pblk * D, pblk * D)],
                    out_hbm.at[pl.ds(((b0 + i) * SEQ + l8) * D, pblk * D)],
                    sem,
                )

        def drain_out(buf, sem):
            pltpu.make_async_copy(
                buf, out_hbm.at[pl.ds(0, bblk * pblk * D)], sem
            ).wait()

        bufs = (buf0, buf1, buf2)
        sems = (sem0, sem1, sem2)
        for q in range(3):
            compute_chunk(q, bufs[q])
            start_out(q, bufs[q], sems[q])

        def triple(kk, _):
            for r in range(3):
                q = kk * 3 + r
                drain_out(bufs[r], sems[r])
                compute_chunk(q, bufs[r])
                start_out(q, bufs[r], sems[r])
            return 0

        lax.fori_loop(1, (nchunk - 1) // 3, triple, 0)
        drain_out(bufs[0], sems[0])
        compute_chunk(nchunk - 1, bufs[0])
        start_out(nchunk - 1, bufs[0], sems[0])
        for r in range(3):
            drain_out(bufs[r], sems[r])

    return sc_gather


def kernel(x, matbert_table, W, b):
    batch, seq = x.shape
    reduced = _reduced_table(matbert_table, W, b)
    pe = jnp.asarray(_PE_CONST)
    sc = _make_sc_kernel(batch)
    return sc(x.T, reduced.reshape(-1), pe).reshape(batch, seq, D)
